# baseline (speedup 1.0000x reference)
import functools

import jax
import jax.numpy as jnp
from jax import lax
from jax.experimental import pallas as pl
from jax.experimental.pallas import tpu as pltpu
from jax.experimental.pallas import tpu_sc as plsc

B, V, E, D = 2, 50000, 150000, 128
VP = 16
NC, NS = 2, 16
NW = NC * NS
TPB = NW // B
EPT = E // TPB
C = 96
NCHT = EPT // C + 1
REM = EPT - (NCHT - 1) * C
EPT_PAD = NCHT * C
NPAIR = (NCHT - 2) // 2
RPT = (B * V) // NW
NCC = RPT // C
REMC = RPT - NCC * C
NCP = NCC // 2
OUTR = B * (V + E)


@functools.cache
def _build_subdiv():
  mesh = plsc.VectorSubcoreMesh(core_axis_name="c", subcore_axis_name="s",
                                num_cores=NC, num_subcores=NS)

  @functools.partial(
      pl.kernel,
      out_type=(
          jax.ShapeDtypeStruct((OUTR, D), jnp.float32),
          jax.ShapeDtypeStruct((OUTR, VP), jnp.float32),
      ),
      mesh=mesh,
      compiler_params=pltpu.CompilerParams(use_tc_tiling_on_sc=False),
      scratch_types=[
          pltpu.VMEM((EPT_PAD,), jnp.int32),
          pltpu.VMEM((EPT_PAD,), jnp.int32),
          pltpu.VMEM((C, D), jnp.float32),
          pltpu.VMEM((C, D), jnp.float32),
          pltpu.VMEM((C, D), jnp.float32),
          pltpu.VMEM((C, D), jnp.float32),
          pltpu.VMEM((C, VP), jnp.float32),
          pltpu.VMEM((C, VP), jnp.float32),
          pltpu.VMEM((C, VP), jnp.float32),
          pltpu.VMEM((C, VP), jnp.float32),
          pltpu.VMEM((C, D), jnp.float32),
          pltpu.VMEM((C, D), jnp.float32),
          pltpu.VMEM((C, VP), jnp.float32),
          pltpu.VMEM((C, VP), jnp.float32),
          pltpu.SemaphoreType.DMA,
          pltpu.SemaphoreType.DMA,
          pltpu.SemaphoreType.DMA,
          pltpu.SemaphoreType.DMA,
          pltpu.SemaphoreType.DMA,
          pltpu.SemaphoreType.DMA,
          pltpu.SemaphoreType.DMA,
          pltpu.SemaphoreType.DMA,
      ],
  )
  def _subdiv(i0_hbm, i1_hbm, feats_hbm, vtab_hbm, outf_hbm, outv_hbm,
              iall0, iall1, fa0, fa1, fb0, fb1, va0, va1, vb0, vb1,
              wo0, wo1, wov0, wov1,
              sg0, sg1, sw0, sw1, sci0, sci1, sco0, sco1):
    wid = lax.axis_index("s") * NC + lax.axis_index("c")
    b = wid // TPB
    j = wid % TPB
    idx_base = wid * EPT_PAD
    mid_base = b * (V + E) + V + j * EPT
    cin_base = b * V + j * RPT
    cout_base = b * (V + E) + j * RPT

    bufs = (
        (fa0, fb0, va0, vb0, wo0, wov0, sg0, sw0),
        (fa1, fb1, va1, vb1, wo1, wov1, sg1, sw1),
    )

    def issue_gathers(g, p):
      fa, fb, va, vb, _, _, sg, _ = bufs[p]
      ia = iall0.at[pl.ds(g * C, C)]
      ib = iall1.at[pl.ds(g * C, C)]
      pltpu.async_copy(feats_hbm.at[ia], fa, sg)
      pltpu.async_copy(feats_hbm.at[ib], fb, sg)
      pltpu.async_copy(vtab_hbm.at[ia], va, sg)
      pltpu.async_copy(vtab_hbm.at[ib], vb, sg)

    def wait_gathers(p):
      fa, fb, va, vb, _, _, sg, _ = bufs[p]
      d = iall0.at[pl.ds(0, C)]
      pltpu.make_async_copy(feats_hbm.at[d], fa, sg).wait()
      pltpu.make_async_copy(feats_hbm.at[d], fb, sg).wait()
      pltpu.make_async_copy(vtab_hbm.at[d], va, sg).wait()
      pltpu.make_async_copy(vtab_hbm.at[d], vb, sg).wait()

    def compute(p):
      fa, fb, va, vb, wo, wov, _, _ = bufs[p]

      def row(r4, carry):
        for dr in range(4):
          r = r4 * 4 + dr
          for k in range(D // 16):
            wo[r, pl.ds(k * 16, 16)] = (
                fa[r, pl.ds(k * 16, 16)] + fb[r, pl.ds(k * 16, 16)]) * 0.5
          wov[r, pl.ds(0, VP)] = (
              va[r, pl.ds(0, VP)] + vb[r, pl.ds(0, VP)]) * 0.5
        return carry

      lax.fori_loop(0, C // 4, row, 0)

    def issue_writes(g, p, nrows):
      _, _, _, _, wo, wov, _, sw = bufs[p]
      pltpu.async_copy(wo.at[pl.ds(0, nrows)],
                       outf_hbm.at[pl.ds(mid_base + g * C, nrows)], sw)
      pltpu.async_copy(wov.at[pl.ds(0, nrows)],
                       outv_hbm.at[pl.ds(mid_base + g * C, nrows)], sw)

    def wait_writes(p, nrows):
      _, _, _, _, wo, wov, _, sw = bufs[p]
      pltpu.make_async_copy(wo.at[pl.ds(0, nrows)],
                            outf_hbm.at[pl.ds(0, nrows)], sw).wait()
      pltpu.make_async_copy(wov.at[pl.ds(0, nrows)],
                            outv_hbm.at[pl.ds(0, nrows)], sw).wait()

    pltpu.sync_copy(i0_hbm.at[pl.ds(idx_base, EPT_PAD)], iall0)
    pltpu.sync_copy(i1_hbm.at[pl.ds(idx_base, EPT_PAD)], iall1)
    issue_gathers(0, 0)
    issue_gathers(1, 1)

    def pair(i, carry):
      for p in (0, 1):
        g = 2 * i + p
        wait_gathers(p)

        @pl.when(g >= 2)
        def _():
          wait_writes(p, C)

        compute(p)
        issue_gathers(g + 2, p)
        issue_writes(g, p, C)
      return carry

    lax.fori_loop(0, NPAIR, pair, 0)

    wait_gathers(0)
    wait_writes(0, C)
    compute(0)
    issue_writes(NCHT - 2, 0, C)
    wait_gathers(1)
    wait_writes(1, C)
    compute(1)
    issue_writes(NCHT - 1, 1, REM)

    cbufs = ((fa0, va0, sci0, sco0), (fa1, va1, sci1, sco1))

    def cpair(jj, carry):
      for p in (0, 1):
        cf, cv, sci, sco = cbufs[p]
        c = 2 * jj + p

        @pl.when(c >= 2)
        def _():
          pltpu.make_async_copy(cf, outf_hbm.at[pl.ds(0, C)], sco).wait()
          pltpu.make_async_copy(cv, outv_hbm.at[pl.ds(0, C)], sco).wait()

        pltpu.async_copy(feats_hbm.at[pl.ds(cin_base + c * C, C)], cf, sci)
        pltpu.async_copy(vtab_hbm.at[pl.ds(cin_base + c * C, C)], cv, sci)
      for p in (0, 1):
        cf, cv, sci, sco = cbufs[p]
        c = 2 * jj + p
        pltpu.make_async_copy(feats_hbm.at[pl.ds(0, C)], cf, sci).wait()
        pltpu.make_async_copy(vtab_hbm.at[pl.ds(0, C)], cv, sci).wait()
        pltpu.async_copy(cf, outf_hbm.at[pl.ds(cout_base + c * C, C)], sco)
        pltpu.async_copy(cv, outv_hbm.at[pl.ds(cout_base + c * C, C)], sco)
      return carry

    lax.fori_loop(0, NCP, cpair, 0)

    for p in (0, 1):
      cf, cv, _, sco = cbufs[p]
      pltpu.make_async_copy(cf, outf_hbm.at[pl.ds(0, C)], sco).wait()
      pltpu.make_async_copy(cv, outv_hbm.at[pl.ds(0, C)], sco).wait()

    pltpu.sync_copy(feats_hbm.at[pl.ds(cin_base + NCC * C, REMC)],
                    fa0.at[pl.ds(0, REMC)])
    pltpu.sync_copy(fa0.at[pl.ds(0, REMC)],
                    outf_hbm.at[pl.ds(cout_base + NCC * C, REMC)])
    pltpu.sync_copy(vtab_hbm.at[pl.ds(cin_base + NCC * C, REMC)],
                    va0.at[pl.ds(0, REMC)])
    pltpu.sync_copy(va0.at[pl.ds(0, REMC)],
                    outv_hbm.at[pl.ds(cout_base + NCC * C, REMC)])

    wait_writes(0, C)
    wait_writes(1, REM)

  return _subdiv


@jax.jit
def kernel(verts, feats, edges):
  offs = (jnp.arange(B, dtype=jnp.int32) * V)[:, None]
  i0 = (edges[:, 0][None, :] + offs).reshape(NW, EPT)
  i1 = (edges[:, 1][None, :] + offs).reshape(NW, EPT)
  i0 = jnp.pad(i0, ((0, 0), (0, EPT_PAD - EPT))).reshape(-1)
  i1 = jnp.pad(i1, ((0, 0), (0, EPT_PAD - EPT))).reshape(-1)
  vtab = jnp.pad(verts.reshape(B * V, 3), ((0, 0), (0, VP - 3)))
  outf, outv = _build_subdiv()(i0, i1, feats, vtab)
  new_verts = outv[:, :3].reshape(B, V + E, 3)
  return new_verts, outf


# speedup vs baseline: 1.5063x; 1.5063x over previous
import functools

import jax
import jax.numpy as jnp
from jax import lax
from jax.experimental import pallas as pl
from jax.experimental.pallas import tpu as pltpu
from jax.experimental.pallas import tpu_sc as plsc

B, V, E, D = 2, 50000, 150000, 128
VP = 16
NC, NS = 2, 16
NW = NC * NS
TPB = NW // B
EPT = E // TPB
C = 96
NCHT = EPT // C + 1
REM = EPT - (NCHT - 1) * C
EPT_PAD = NCHT * C
NPAIR = (NCHT - 2) // 2
RPT = (B * V) // NW
NCC = RPT // C
REMC = RPT - NCC * C
NCP = NCC // 2
OUTR = B * (V + E)


@functools.cache
def _build_subdiv():
  mesh = plsc.VectorSubcoreMesh(core_axis_name="c", subcore_axis_name="s",
                                num_cores=NC, num_subcores=NS)

  @functools.partial(
      pl.kernel,
      out_type=(
          jax.ShapeDtypeStruct((OUTR, D), jnp.float32),
          jax.ShapeDtypeStruct((OUTR, VP), jnp.float32),
      ),
      mesh=mesh,
      compiler_params=pltpu.CompilerParams(use_tc_tiling_on_sc=False),
      scratch_types=[
          pltpu.VMEM((EPT_PAD,), jnp.int32),
          pltpu.VMEM((EPT_PAD,), jnp.int32),
          pltpu.VMEM((C, D), jnp.float32),
          pltpu.VMEM((C, D), jnp.float32),
          pltpu.VMEM((C, D), jnp.float32),
          pltpu.VMEM((C, D), jnp.float32),
          pltpu.VMEM((C, VP), jnp.float32),
          pltpu.VMEM((C, VP), jnp.float32),
          pltpu.VMEM((C, VP), jnp.float32),
          pltpu.VMEM((C, VP), jnp.float32),
          pltpu.VMEM((C, D), jnp.float32),
          pltpu.VMEM((C, D), jnp.float32),
          pltpu.VMEM((C, VP), jnp.float32),
          pltpu.VMEM((C, VP), jnp.float32),
          pltpu.SemaphoreType.DMA,
          pltpu.SemaphoreType.DMA,
          pltpu.SemaphoreType.DMA,
          pltpu.SemaphoreType.DMA,
          pltpu.SemaphoreType.DMA,
          pltpu.SemaphoreType.DMA,
          pltpu.SemaphoreType.DMA,
          pltpu.SemaphoreType.DMA,
      ],
  )
  def _subdiv(i0_hbm, i1_hbm, feats_hbm, vtab_hbm, outf_hbm, outv_hbm,
              iall0, iall1, fa0, fa1, fb0, fb1, va0, va1, vb0, vb1,
              wo0, wo1, wov0, wov1,
              sg0, sg1, sw0, sw1, sci0, sci1, sco0, sco1):
    wid = lax.axis_index("s") * NC + lax.axis_index("c")
    b = wid // TPB
    j = wid % TPB
    idx_base = wid * EPT_PAD
    mid_base = b * (V + E) + V + j * EPT
    cin_base = b * V + j * RPT
    cout_base = b * (V + E) + j * RPT

    bufs = (
        (fa0, fb0, va0, vb0, wo0, wov0, sg0, sw0),
        (fa1, fb1, va1, vb1, wo1, wov1, sg1, sw1),
    )

    def issue_gathers(g, p):
      fa, fb, va, vb, _, _, sg, _ = bufs[p]
      ia = iall0.at[pl.ds(g * C, C)]
      ib = iall1.at[pl.ds(g * C, C)]
      pltpu.async_copy(feats_hbm.at[ia], fa, sg)
      pltpu.async_copy(feats_hbm.at[ib], fb, sg)
      pltpu.async_copy(vtab_hbm.at[ia], va, sg)
      pltpu.async_copy(vtab_hbm.at[ib], vb, sg)

    def wait_gathers(p):
      fa, fb, va, vb, _, _, sg, _ = bufs[p]
      d = iall0.at[pl.ds(0, C)]
      pltpu.make_async_copy(feats_hbm.at[d], fa, sg).wait()
      pltpu.make_async_copy(feats_hbm.at[d], fb, sg).wait()
      pltpu.make_async_copy(vtab_hbm.at[d], va, sg).wait()
      pltpu.make_async_copy(vtab_hbm.at[d], vb, sg).wait()

    def compute(p):
      fa, fb, va, vb, wo, wov, _, _ = bufs[p]

      @plsc.parallel_loop(0, C, step=1, unroll=4)
      def row(r):
        for k in range(D // 16):
          wo[r, pl.ds(k * 16, 16)] = (
              fa[r, pl.ds(k * 16, 16)] + fb[r, pl.ds(k * 16, 16)]) * 0.5
        wov[r, pl.ds(0, VP)] = (
            va[r, pl.ds(0, VP)] + vb[r, pl.ds(0, VP)]) * 0.5

    def issue_writes(g, p, nrows):
      _, _, _, _, wo, wov, _, sw = bufs[p]
      pltpu.async_copy(wo.at[pl.ds(0, nrows)],
                       outf_hbm.at[pl.ds(mid_base + g * C, nrows)], sw)
      pltpu.async_copy(wov.at[pl.ds(0, nrows)],
                       outv_hbm.at[pl.ds(mid_base + g * C, nrows)], sw)

    def wait_writes(p, nrows):
      _, _, _, _, wo, wov, _, sw = bufs[p]
      pltpu.make_async_copy(wo.at[pl.ds(0, nrows)],
                            outf_hbm.at[pl.ds(0, nrows)], sw).wait()
      pltpu.make_async_copy(wov.at[pl.ds(0, nrows)],
                            outv_hbm.at[pl.ds(0, nrows)], sw).wait()

    pltpu.sync_copy(i0_hbm.at[pl.ds(idx_base, EPT_PAD)], iall0)
    pltpu.sync_copy(i1_hbm.at[pl.ds(idx_base, EPT_PAD)], iall1)
    issue_gathers(0, 0)
    issue_gathers(1, 1)

    def pair(i, carry):
      for p in (0, 1):
        g = 2 * i + p
        wait_gathers(p)

        @pl.when(g >= 2)
        def _():
          wait_writes(p, C)

        compute(p)
        issue_gathers(g + 2, p)
        issue_writes(g, p, C)
      return carry

    lax.fori_loop(0, NPAIR, pair, 0)

    wait_gathers(0)
    wait_writes(0, C)
    compute(0)
    issue_writes(NCHT - 2, 0, C)
    wait_gathers(1)
    wait_writes(1, C)
    compute(1)
    issue_writes(NCHT - 1, 1, REM)

    cbufs = ((fa0, va0, sci0, sco0), (fa1, va1, sci1, sco1))

    def cpair(jj, carry):
      for p in (0, 1):
        cf, cv, sci, sco = cbufs[p]
        c = 2 * jj + p

        @pl.when(c >= 2)
        def _():
          pltpu.make_async_copy(cf, outf_hbm.at[pl.ds(0, C)], sco).wait()
          pltpu.make_async_copy(cv, outv_hbm.at[pl.ds(0, C)], sco).wait()

        pltpu.async_copy(feats_hbm.at[pl.ds(cin_base + c * C, C)], cf, sci)
        pltpu.async_copy(vtab_hbm.at[pl.ds(cin_base + c * C, C)], cv, sci)
      for p in (0, 1):
        cf, cv, sci, sco = cbufs[p]
        c = 2 * jj + p
        pltpu.make_async_copy(feats_hbm.at[pl.ds(0, C)], cf, sci).wait()
        pltpu.make_async_copy(vtab_hbm.at[pl.ds(0, C)], cv, sci).wait()
        pltpu.async_copy(cf, outf_hbm.at[pl.ds(cout_base + c * C, C)], sco)
        pltpu.async_copy(cv, outv_hbm.at[pl.ds(cout_base + c * C, C)], sco)
      return carry

    lax.fori_loop(0, NCP, cpair, 0)

    for p in (0, 1):
      cf, cv, _, sco = cbufs[p]
      pltpu.make_async_copy(cf, outf_hbm.at[pl.ds(0, C)], sco).wait()
      pltpu.make_async_copy(cv, outv_hbm.at[pl.ds(0, C)], sco).wait()

    pltpu.sync_copy(feats_hbm.at[pl.ds(cin_base + NCC * C, REMC)],
                    fa0.at[pl.ds(0, REMC)])
    pltpu.sync_copy(fa0.at[pl.ds(0, REMC)],
                    outf_hbm.at[pl.ds(cout_base + NCC * C, REMC)])
    pltpu.sync_copy(vtab_hbm.at[pl.ds(cin_base + NCC * C, REMC)],
                    va0.at[pl.ds(0, REMC)])
    pltpu.sync_copy(va0.at[pl.ds(0, REMC)],
                    outv_hbm.at[pl.ds(cout_base + NCC * C, REMC)])

    wait_writes(0, C)
    wait_writes(1, REM)

  return _subdiv


@jax.jit
def kernel(verts, feats, edges):
  offs = (jnp.arange(B, dtype=jnp.int32) * V)[:, None]
  i0 = (edges[:, 0][None, :] + offs).reshape(NW, EPT)
  i1 = (edges[:, 1][None, :] + offs).reshape(NW, EPT)
  i0 = jnp.pad(i0, ((0, 0), (0, EPT_PAD - EPT))).reshape(-1)
  i1 = jnp.pad(i1, ((0, 0), (0, EPT_PAD - EPT))).reshape(-1)
  vtab = jnp.pad(verts.reshape(B * V, 3), ((0, 0), (0, VP - 3)))
  outf, outv = _build_subdiv()(i0, i1, feats, vtab)
  new_verts = outv[:, :3].reshape(B, V + E, 3)
  return new_verts, outf


# speedup vs baseline: 1.5712x; 1.0431x over previous
import functools

import jax
import jax.numpy as jnp
from jax import lax
from jax.experimental import pallas as pl
from jax.experimental.pallas import tpu as pltpu
from jax.experimental.pallas import tpu_sc as plsc

B, V, E, D = 2, 50000, 150000, 128
VP = 16
NC, NS = 2, 16
NW = NC * NS
TPB = NW // B
EPT = 9376
C = 96
NCHT = EPT // C + 1
REM = EPT - (NCHT - 1) * C
EPT_PAD = NCHT * C
NPAIR = (NCHT - 2) // 2
RPT = 3128
NCC = RPT // C
REMC = RPT - NCC * C
NCP = NCC // 2
OUTR = B * (V + E)


@functools.cache
def _build(width, tiled):
  mesh = plsc.VectorSubcoreMesh(core_axis_name="c", subcore_axis_name="s",
                                num_cores=NC, num_subcores=NS)
  params = (pltpu.CompilerParams() if tiled else
            pltpu.CompilerParams(use_tc_tiling_on_sc=False))

  @functools.partial(
      pl.kernel,
      out_type=jax.ShapeDtypeStruct((OUTR, width), jnp.float32),
      mesh=mesh,
      compiler_params=params,
      scratch_types=[
          pltpu.VMEM((EPT_PAD,), jnp.int32),
          pltpu.VMEM((EPT_PAD,), jnp.int32),
          pltpu.VMEM((C, width), jnp.float32),
          pltpu.VMEM((C, width), jnp.float32),
          pltpu.VMEM((C, width), jnp.float32),
          pltpu.VMEM((C, width), jnp.float32),
          pltpu.VMEM((C, width), jnp.float32),
          pltpu.VMEM((C, width), jnp.float32),
          pltpu.SemaphoreType.DMA,
          pltpu.SemaphoreType.DMA,
          pltpu.SemaphoreType.DMA,
          pltpu.SemaphoreType.DMA,
          pltpu.SemaphoreType.DMA,
          pltpu.SemaphoreType.DMA,
          pltpu.SemaphoreType.DMA,
          pltpu.SemaphoreType.DMA,
      ],
  )
  def _run(i0_hbm, i1_hbm, tab_hbm, out_hbm,
           iall0, iall1, fa0, fa1, fb0, fb1, wo0, wo1,
           sg0, sg1, sw0, sw1, sci0, sci1, sco0, sco1):
    wid = lax.axis_index("s") * NC + lax.axis_index("c")
    b = wid // TPB
    j = wid % TPB
    idx_base = wid * EPT_PAD
    start = jnp.minimum(j * EPT, E - EPT)
    mid_base = b * (V + E) + V + start
    cstart = jnp.minimum(j * RPT, V - RPT)
    cin_base = b * V + cstart
    cout_base = b * (V + E) + cstart

    bufs = ((fa0, fb0, wo0, sg0, sw0), (fa1, fb1, wo1, sg1, sw1))

    def issue_gathers(g, p):
      fa, fb, _, sg, _ = bufs[p]
      pltpu.async_copy(tab_hbm.at[iall0.at[pl.ds(g * C, C)]], fa, sg)
      pltpu.async_copy(tab_hbm.at[iall1.at[pl.ds(g * C, C)]], fb, sg)

    def wait_gathers(p):
      fa, fb, _, sg, _ = bufs[p]
      d = iall0.at[pl.ds(0, C)]
      pltpu.make_async_copy(tab_hbm.at[d], fa, sg).wait()
      pltpu.make_async_copy(tab_hbm.at[d], fb, sg).wait()

    def compute(p):
      fa, fb, wo, _, _ = bufs[p]

      @plsc.parallel_loop(0, C, step=1, unroll=4)
      def row(r):
        for k in range(width // 16):
          wo[r, pl.ds(k * 16, 16)] = (
              fa[r, pl.ds(k * 16, 16)] + fb[r, pl.ds(k * 16, 16)]) * 0.5

    def issue_writes(g, p, nrows):
      _, _, wo, _, sw = bufs[p]
      pltpu.async_copy(wo.at[pl.ds(0, nrows)],
                       out_hbm.at[pl.ds(mid_base + g * C, nrows)], sw)

    def wait_writes(p, nrows):
      _, _, wo, _, sw = bufs[p]
      pltpu.make_async_copy(wo.at[pl.ds(0, nrows)],
                            out_hbm.at[pl.ds(0, nrows)], sw).wait()

    pltpu.sync_copy(i0_hbm.at[pl.ds(idx_base, EPT_PAD)], iall0)
    pltpu.sync_copy(i1_hbm.at[pl.ds(idx_base, EPT_PAD)], iall1)
    issue_gathers(0, 0)
    issue_gathers(1, 1)

    def pair(i, carry):
      for p in (0, 1):
        g = 2 * i + p
        wait_gathers(p)

        @pl.when(g >= 2)
        def _():
          wait_writes(p, C)

        compute(p)
        issue_gathers(g + 2, p)
        issue_writes(g, p, C)
      return carry

    lax.fori_loop(0, NPAIR, pair, 0)

    wait_gathers(0)
    wait_writes(0, C)
    compute(0)
    issue_writes(NCHT - 2, 0, C)
    wait_gathers(1)
    wait_writes(1, C)
    compute(1)
    issue_writes(NCHT - 1, 1, REM)

    cbufs = ((fa0, sci0, sco0), (fa1, sci1, sco1))

    def cpair(jj, carry):
      for p in (0, 1):
        cf, sci, sco = cbufs[p]
        c = 2 * jj + p

        @pl.when(c >= 2)
        def _():
          pltpu.make_async_copy(cf, out_hbm.at[pl.ds(0, C)], sco).wait()

        pltpu.async_copy(tab_hbm.at[pl.ds(cin_base + c * C, C)], cf, sci)
      for p in (0, 1):
        cf, sci, sco = cbufs[p]
        c = 2 * jj + p
        pltpu.make_async_copy(tab_hbm.at[pl.ds(0, C)], cf, sci).wait()
        pltpu.async_copy(cf, out_hbm.at[pl.ds(cout_base + c * C, C)], sco)
      return carry

    lax.fori_loop(0, NCP, cpair, 0)

    for p in (0, 1):
      cf, _, sco = cbufs[p]
      pltpu.make_async_copy(cf, out_hbm.at[pl.ds(0, C)], sco).wait()

    pltpu.sync_copy(tab_hbm.at[pl.ds(cin_base + NCC * C, REMC)],
                    fa0.at[pl.ds(0, REMC)])
    pltpu.sync_copy(fa0.at[pl.ds(0, REMC)],
                    out_hbm.at[pl.ds(cout_base + NCC * C, REMC)])

    wait_writes(0, C)
    wait_writes(1, REM)

  return _run


def _tile_indices(e, offs):
  main = e[:(TPB - 1) * EPT].reshape(TPB - 1, EPT)
  last = e[E - EPT:][None, :]
  per_tile = jnp.concatenate([main, last], axis=0)
  both = per_tile[None, :, :] + offs[:, None, None]
  return jnp.pad(both, ((0, 0), (0, 0), (0, EPT_PAD - EPT))).reshape(-1)


@jax.jit
def kernel(verts, feats, edges):
  offs = jnp.arange(B, dtype=jnp.int32) * V
  i0 = _tile_indices(edges[:, 0], offs)
  i1 = _tile_indices(edges[:, 1], offs)
  vtab = jnp.pad(verts.reshape(B * V, 3), ((0, 0), (0, VP - 3)))
  outf = _build(D, True)(i0, i1, feats)
  outv = _build(VP, False)(i0, i1, vtab)
  new_verts = outv[:, :3].reshape(B, V + E, 3)
  return new_verts, outf
